# baseline (speedup 1.0000x reference)
import functools

import jax
import jax.numpy as jnp
from jax import lax
from jax.experimental import pallas as pl
from jax.experimental.pallas import tpu as pltpu
from jax.experimental.pallas import tpu_sc as plsc

EMBED_DIM = 16
FIELD_OFFSET = 1000000
NUM_WORKERS = 32
CHUNK = 128


def _sc_gather(x_flat, table):
    tot = x_flat.shape[0]
    per_w = tot // NUM_WORKERS
    n_chunks = per_w // CHUNK
    mesh = plsc.VectorSubcoreMesh(core_axis_name="c", subcore_axis_name="s")

    @functools.partial(
        pl.kernel,
        mesh=mesh,
        out_type=jax.ShapeDtypeStruct((tot, EMBED_DIM), jnp.float32),
        compiler_params=pltpu.CompilerParams(use_tc_tiling_on_sc=False),
        scratch_types=[
            pltpu.VMEM((n_chunks, CHUNK), jnp.int32),
            pltpu.VMEM((per_w, EMBED_DIM), jnp.float32),
            pltpu.SemaphoreType.DMA,
        ],
    )
    def k(x_hbm, table_hbm, out_hbm, idx_v, rows_v, sem):
        wid = lax.axis_index("s") * 2 + lax.axis_index("c")
        base = wid * per_w
        for j in range(n_chunks):
            pltpu.sync_copy(x_hbm.at[pl.ds(base + j * CHUNK, CHUNK)], idx_v.at[j])
        offs = (lax.iota(jnp.int32, 16) % 2) * FIELD_OFFSET
        for j in range(n_chunks):
            for i in range(CHUNK // 16):
                sl = pl.ds(i * 16, 16)
                idx_v[j, sl] = idx_v[j, sl] + offs
        copies = [
            pltpu.async_copy(
                table_hbm.at[idx_v.at[j]],
                rows_v.at[pl.ds(j * CHUNK, CHUNK)],
                sem,
            )
            for j in range(n_chunks)
        ]
        for c in copies:
            c.wait()
        pltpu.sync_copy(rows_v, out_hbm.at[pl.ds(base, per_w)])

    return k(x_flat, table)


def _tc_mlp(emb, W1, b1, W2, b2, W3, b3, Wfc, bfc):
    B = emb.shape[0]
    blk = 2048
    grid = (B // blk,)

    def body(e_ref, w1_ref, b1_ref, w2_ref, b2_ref, w3_ref, b3_ref,
             wfc_ref, bfc_ref, o_ref):
        e = e_ref[...]
        h = jnp.maximum(
            jnp.dot(e, w1_ref[...], preferred_element_type=jnp.float32)
            + b1_ref[...], 0.0)
        h = jnp.maximum(
            jnp.dot(h, w2_ref[...], preferred_element_type=jnp.float32)
            + b2_ref[...], 0.0)
        h = jnp.maximum(
            jnp.dot(h, w3_ref[...], preferred_element_type=jnp.float32)
            + b3_ref[...], 0.0)
        gmf = e[:, :EMBED_DIM] * e[:, EMBED_DIM:]
        cat = jnp.concatenate([gmf, h], axis=1)
        wfc = wfc_ref[...]
        o_ref[...] = jnp.sum(cat * wfc, axis=1) + bfc_ref[0]

    rep = lambda shape: pl.BlockSpec(shape, lambda i: tuple(0 for _ in shape))
    return pl.pallas_call(
        body,
        grid=grid,
        in_specs=[
            pl.BlockSpec((blk, emb.shape[1]), lambda i: (i, 0)),
            rep(W1.shape),
            rep((1, b1.shape[0])),
            rep(W2.shape),
            rep((1, b2.shape[0])),
            rep(W3.shape),
            rep((1, b3.shape[0])),
            rep((1, Wfc.shape[0])),
            rep((1,)),
        ],
        out_specs=pl.BlockSpec((blk,), lambda i: (i,)),
        out_shape=jax.ShapeDtypeStruct((B,), jnp.float32),
    )(emb, W1, b1.reshape(1, -1), W2, b2.reshape(1, -1), W3,
      b3.reshape(1, -1), Wfc.reshape(1, -1), bfc)


def kernel(x, table, W1, b1, W2, b2, W3, b3, Wfc, bfc):
    B = x.shape[0]
    x_flat = x.reshape(-1).astype(jnp.int32)
    rows = _sc_gather(x_flat, table)
    emb = rows.reshape(B, 2 * EMBED_DIM)
    return _tc_mlp(emb, W1, b1, W2, b2, W3, b3, Wfc, bfc)


# speedup vs baseline: 10.1472x; 10.1472x over previous
import functools

import jax
import jax.numpy as jnp
from jax import lax
from jax.experimental import pallas as pl
from jax.experimental.pallas import tpu as pltpu
from jax.experimental.pallas import tpu_sc as plsc

EMBED_DIM = 16
NROWS = 2000000
FIELD_OFFSET = 1000000
NUM_WORKERS = 32
LANE_TILES = NROWS // 128
CHUNK = 128
FIRE = 8


def _table_phys_flat(table):
    t = table.T.reshape(2, 8, LANE_TILES, 128)
    return t.transpose(0, 2, 1, 3).reshape(-1)


def _sc_gather(x_flat, tflat):
    tot = x_flat.shape[0]
    per_w = tot // NUM_WORKERS
    n_words = per_w * EMBED_DIM
    n_chunks = n_words // CHUNK
    mesh = plsc.VectorSubcoreMesh(core_axis_name="c", subcore_axis_name="s")

    @functools.partial(
        pl.kernel,
        mesh=mesh,
        out_type=jax.ShapeDtypeStruct((tot * EMBED_DIM,), jnp.float32),
        compiler_params=pltpu.CompilerParams(
            use_tc_tiling_on_sc=False, needs_layout_passes=False),
        scratch_types=[
            pltpu.VMEM((per_w,), jnp.int32),
            pltpu.VMEM((n_words,), jnp.int32),
            pltpu.VMEM((n_words,), jnp.float32),
            pltpu.SemaphoreType.DMA,
        ],
    )
    def k(x_hbm, table_hbm, out_hbm, x_v, w_idx, rows_v, sem):
        wid = lax.axis_index("s") * 2 + lax.axis_index("c")
        base = wid * per_w
        pltpu.sync_copy(x_hbm.at[pl.ds(base, per_w)], x_v)

        offs = (lax.iota(jnp.int32, 16) % 2) * FIELD_OFFSET
        iota16x = lax.iota(jnp.int32, 16) * 16

        def build(g, _):
            r16 = x_v[pl.ds(g * 16, 16)] + offs
            rp = ((r16 >> 7) << 10) + (r16 & 127)
            p0 = g * 256
            for j in range(EMBED_DIM):
                jconst = (j // 8) * (LANE_TILES * 1024) + (j % 8) * 128
                plsc.store_scatter(w_idx, [iota16x + (p0 + j)], rp + jconst)
            return 0

        lax.fori_loop(0, per_w // 16, build, 0)

        def gather(step, _):
            c0 = step * FIRE
            copies = [
                pltpu.async_copy(
                    table_hbm.at[w_idx.at[pl.ds((c0 + f) * CHUNK, CHUNK)]],
                    rows_v.at[pl.ds((c0 + f) * CHUNK, CHUNK)],
                    sem,
                )
                for f in range(FIRE)
            ]
            for c in copies:
                c.wait()
            return 0

        lax.fori_loop(0, n_chunks // FIRE, gather, 0)
        pltpu.sync_copy(rows_v, out_hbm.at[pl.ds(base * EMBED_DIM, n_words)])

    return k(x_flat, tflat)


def _tc_mlp(emb, W1, b1, W2, b2, W3, b3, Wfc, bfc):
    B = emb.shape[0]
    blk = 2048
    grid = (B // blk,)

    def body(e_ref, w1_ref, b1_ref, w2_ref, b2_ref, w3_ref, b3_ref,
             wfc_ref, bfc_ref, o_ref):
        e = e_ref[...]
        h = jnp.maximum(
            jnp.dot(e, w1_ref[...], preferred_element_type=jnp.float32)
            + b1_ref[...], 0.0)
        h = jnp.maximum(
            jnp.dot(h, w2_ref[...], preferred_element_type=jnp.float32)
            + b2_ref[...], 0.0)
        h = jnp.maximum(
            jnp.dot(h, w3_ref[...], preferred_element_type=jnp.float32)
            + b3_ref[...], 0.0)
        gmf = e[:, :EMBED_DIM] * e[:, EMBED_DIM:]
        cat = jnp.concatenate([gmf, h], axis=1)
        wfc = wfc_ref[...]
        o_ref[...] = jnp.sum(cat * wfc, axis=1) + bfc_ref[0]

    rep = lambda shape: pl.BlockSpec(shape, lambda i: tuple(0 for _ in shape))
    return pl.pallas_call(
        body,
        grid=grid,
        in_specs=[
            pl.BlockSpec((blk, emb.shape[1]), lambda i: (i, 0)),
            rep(W1.shape),
            rep((1, b1.shape[0])),
            rep(W2.shape),
            rep((1, b2.shape[0])),
            rep(W3.shape),
            rep((1, b3.shape[0])),
            rep((1, Wfc.shape[0])),
            rep((1,)),
        ],
        out_specs=pl.BlockSpec((blk,), lambda i: (i,)),
        out_shape=jax.ShapeDtypeStruct((B,), jnp.float32),
    )(emb, W1, b1.reshape(1, -1), W2, b2.reshape(1, -1), W3,
      b3.reshape(1, -1), Wfc.reshape(1, -1), bfc)


def kernel(x, table, W1, b1, W2, b2, W3, b3, Wfc, bfc):
    B = x.shape[0]
    x_flat = x.reshape(-1).astype(jnp.int32)
    rows = _sc_gather(x_flat, _table_phys_flat(table))
    emb = rows.reshape(B, 2 * EMBED_DIM)
    return _tc_mlp(emb, W1, b1, W2, b2, W3, b3, Wfc, bfc)


# speedup vs baseline: 14.0847x; 1.3880x over previous
import functools

import jax
import jax.numpy as jnp
from jax import lax
from jax.experimental import pallas as pl
from jax.experimental.pallas import tpu as pltpu
from jax.experimental.pallas import tpu_sc as plsc

EMBED_DIM = 16
NROWS = 2000000
FIELD_OFFSET = 1000000
NUM_WORKERS = 32
LANE_TILES = NROWS // 128
CHUNK = 128


def _table_phys_flat(table):
    t = table.T.reshape(2, 8, LANE_TILES, 128)
    return t.transpose(0, 2, 1, 3).reshape(-1)


def _sc_gather_t(x, tflat):
    B = x.shape[0]
    per_w = B // NUM_WORKERS
    n_planes = 2 * EMBED_DIM
    n_words = per_w * n_planes
    n_chunks = n_words // CHUNK
    chunks_per_plane = per_w // CHUNK
    col_tiles = B // 128
    mesh = plsc.VectorSubcoreMesh(core_axis_name="c", subcore_axis_name="s")

    @functools.partial(
        pl.kernel,
        mesh=mesh,
        out_type=jax.ShapeDtypeStruct((B * n_planes // 128, 128), jnp.float32),
        compiler_params=pltpu.CompilerParams(
            use_tc_tiling_on_sc=False, needs_layout_passes=False),
        scratch_types=[
            pltpu.VMEM((per_w, 2), jnp.int32),
            pltpu.VMEM((2, per_w), jnp.int32),
            pltpu.VMEM((n_words,), jnp.int32),
            pltpu.VMEM((n_planes, per_w), jnp.float32),
            pltpu.SemaphoreType.DMA,
        ],
    )
    def k(x_hbm, table_hbm, out_hbm, x_v, rp_v, w_idx, rows_v, sem):
        wid = lax.axis_index("s") * 2 + lax.axis_index("c")
        row0 = wid * per_w
        pltpu.sync_copy(x_hbm.at[pl.ds(row0, per_w), :], x_v)

        iota = lax.iota(jnp.int32, 16)
        zeros = iota * 0

        def build_rp(g, _):
            rows16 = g * 16 + iota
            for f in range(2):
                r16 = plsc.load_gather(x_v, [rows16, zeros + f]) + f * FIELD_OFFSET
                rp_v[f, pl.ds(g * 16, 16)] = ((r16 >> 7) << 10) + (r16 & 127)
            return 0

        lax.fori_loop(0, per_w // 16, build_rp, 0)

        def fire(c, _):
            p = c // chunks_per_plane
            b0 = (c % chunks_per_plane) * CHUNK
            f = p // EMBED_DIM
            j = p % EMBED_DIM
            jconst = (j // 8) * (LANE_TILES * 1024) + (j % 8) * 128
            for g in range(CHUNK // 16):
                sl = pl.ds(b0 + g * 16, 16)
                w_idx[pl.ds(c * CHUNK + g * 16, 16)] = rp_v[f, sl] + jconst
            pltpu.async_copy(
                table_hbm.at[w_idx.at[pl.ds(c * CHUNK, CHUNK)]],
                rows_v.at[p, pl.ds(b0, CHUNK)],
                sem,
            )
            return 0

        lax.fori_loop(0, n_chunks, fire, 0)
        for p in range(n_planes):
            pltpu.make_async_copy(
                table_hbm.at[pl.ds(0, per_w)], rows_v.at[p, :], sem
            ).wait()

        for R in range(n_planes // 8):
            for c in range(per_w // 128):
                t = R * col_tiles + wid * (per_w // 128) + c
                pltpu.sync_copy(
                    rows_v.at[pl.ds(8 * R, 8), pl.ds(128 * c, 128)],
                    out_hbm.at[pl.ds(8 * t, 8), :],
                )

    return k(x, tflat)


def _tc_mlp_t(embT, w1t, b1c, w2t, b2c, w3t, b3c, wg, wh, bfc):
    B = embT.shape[1]
    blk = 2048
    grid = (B // blk,)

    def body(e_ref, w1_ref, b1_ref, w2_ref, b2_ref, w3_ref, b3_ref,
             wg_ref, wh_ref, bfc_ref, o_ref):
        e = e_ref[...]
        h = jnp.maximum(
            jnp.dot(w1_ref[...], e, preferred_element_type=jnp.float32)
            + b1_ref[...], 0.0)
        h = jnp.maximum(
            jnp.dot(w2_ref[...], h, preferred_element_type=jnp.float32)
            + b2_ref[...], 0.0)
        h = jnp.maximum(
            jnp.dot(w3_ref[...], h, preferred_element_type=jnp.float32)
            + b3_ref[...], 0.0)
        gmf = e[:EMBED_DIM, :] * e[EMBED_DIM:, :]
        out = (jnp.dot(wg_ref[...], gmf, preferred_element_type=jnp.float32)
               + jnp.dot(wh_ref[...], h, preferred_element_type=jnp.float32)
               + bfc_ref[0])
        o_ref[...] = out[0]

    rep = lambda shape: pl.BlockSpec(shape, lambda i: tuple(0 for _ in shape))
    return pl.pallas_call(
        body,
        grid=grid,
        in_specs=[
            pl.BlockSpec((embT.shape[0], blk), lambda i: (0, i)),
            rep(w1t.shape),
            rep(b1c.shape),
            rep(w2t.shape),
            rep(b2c.shape),
            rep(w3t.shape),
            rep(b3c.shape),
            rep(wg.shape),
            rep(wh.shape),
            rep((1,)),
        ],
        out_specs=pl.BlockSpec((blk,), lambda i: (i,)),
        out_shape=jax.ShapeDtypeStruct((B,), jnp.float32),
    )(embT, w1t, b1c, w2t, b2c, w3t, b3c, wg, wh, bfc)


def kernel(x, table, W1, b1, W2, b2, W3, b3, Wfc, bfc):
    B = x.shape[0]
    out2d = _sc_gather_t(x.astype(jnp.int32), _table_phys_flat(table))
    embT = (out2d.reshape(4, B // 128, 8, 128)
            .transpose(0, 2, 1, 3)
            .reshape(2 * EMBED_DIM, B))
    return _tc_mlp_t(
        embT,
        W1.T, b1.reshape(-1, 1),
        W2.T, b2.reshape(-1, 1),
        W3.T, b3.reshape(-1, 1),
        Wfc[:EMBED_DIM, :].T, Wfc[EMBED_DIM:, :].T, bfc,
    )


# speedup vs baseline: 16.2070x; 1.1507x over previous
import functools

import jax
import jax.numpy as jnp
from jax import lax
from jax.experimental import pallas as pl
from jax.experimental.pallas import tpu as pltpu
from jax.experimental.pallas import tpu_sc as plsc

EMBED_DIM = 16
NROWS = 2000000
FIELD_OFFSET = 1000000
NUM_WORKERS = 32
LANE_TILES = NROWS // 128
CHUNK = 128


def _table_phys_flat(table):
    t = table.T.reshape(2, 8, LANE_TILES, 128)
    return t.transpose(0, 2, 1, 3).reshape(-1)


def _sc_gather_t(x, tflat):
    B = x.shape[0] // 2
    per_w = B // NUM_WORKERS
    n_planes = 2 * EMBED_DIM
    n_words = per_w * n_planes
    n_chunks = n_words // CHUNK
    chunks_per_plane = per_w // CHUNK
    col_tiles = B // 128
    mesh = plsc.VectorSubcoreMesh(core_axis_name="c", subcore_axis_name="s")

    @functools.partial(
        pl.kernel,
        mesh=mesh,
        out_type=jax.ShapeDtypeStruct((B * n_planes // 128, 128), jnp.float32),
        compiler_params=pltpu.CompilerParams(
            use_tc_tiling_on_sc=False, needs_layout_passes=False),
        scratch_types=[
            pltpu.VMEM((2 * per_w,), jnp.int32),
            pltpu.VMEM((2, per_w), jnp.int32),
            pltpu.VMEM((n_words,), jnp.int32),
            pltpu.VMEM((n_planes, per_w), jnp.float32),
            pltpu.SemaphoreType.DMA,
        ],
    )
    def k(x_hbm, table_hbm, out_hbm, x_v, rp_v, w_idx, rows_v, sem):
        wid = lax.axis_index("s") * 2 + lax.axis_index("c")
        row0 = wid * per_w
        pltpu.sync_copy(x_hbm.at[pl.ds(2 * row0, 2 * per_w)], x_v)

        iota = lax.iota(jnp.int32, 16)

        def build_rp(g, _):
            rows16 = (g * 16 + iota) * 2
            for f in range(2):
                r16 = plsc.load_gather(x_v, [rows16 + f]) + f * FIELD_OFFSET
                rp_v[f, pl.ds(g * 16, 16)] = ((r16 >> 7) << 10) + (r16 & 127)
            return 0

        lax.fori_loop(0, per_w // 16, build_rp, 0)

        def fire(c, _):
            p = c // chunks_per_plane
            b0 = (c % chunks_per_plane) * CHUNK
            f = p // EMBED_DIM
            j = p % EMBED_DIM
            jconst = (j // 8) * (LANE_TILES * 1024) + (j % 8) * 128
            for g in range(CHUNK // 16):
                sl = pl.ds(b0 + g * 16, 16)
                w_idx[pl.ds(c * CHUNK + g * 16, 16)] = rp_v[f, sl] + jconst
            pltpu.async_copy(
                table_hbm.at[w_idx.at[pl.ds(c * CHUNK, CHUNK)]],
                rows_v.at[p, pl.ds(b0, CHUNK)],
                sem,
            )
            return 0

        lax.fori_loop(0, n_chunks, fire, 0)
        for p in range(n_planes):
            pltpu.make_async_copy(
                table_hbm.at[pl.ds(0, per_w)], rows_v.at[p, :], sem
            ).wait()

        for R in range(n_planes // 8):
            for c in range(per_w // 128):
                t = R * col_tiles + wid * (per_w // 128) + c
                pltpu.sync_copy(
                    rows_v.at[pl.ds(8 * R, 8), pl.ds(128 * c, 128)],
                    out_hbm.at[pl.ds(8 * t, 8), :],
                )

    return k(x, tflat)


def _tc_mlp_t(embT, w1t, b1c, w2t, b2c, w3t, b3c, wg, wh, bfc):
    B = embT.shape[1]
    blk = 4096
    grid = (B // blk,)

    def body(e_ref, w1_ref, b1_ref, w2_ref, b2_ref, w3_ref, b3_ref,
             wg_ref, wh_ref, bfc_ref, o_ref):
        e = e_ref[...]
        h = jnp.maximum(
            jnp.dot(w1_ref[...], e, preferred_element_type=jnp.float32)
            + b1_ref[...], 0.0)
        h = jnp.maximum(
            jnp.dot(w2_ref[...], h, preferred_element_type=jnp.float32)
            + b2_ref[...], 0.0)
        h = jnp.maximum(
            jnp.dot(w3_ref[...], h, preferred_element_type=jnp.float32)
            + b3_ref[...], 0.0)
        gmf = e[:EMBED_DIM, :] * e[EMBED_DIM:, :]
        out = (jnp.dot(wg_ref[...], gmf, preferred_element_type=jnp.float32)
               + jnp.dot(wh_ref[...], h, preferred_element_type=jnp.float32)
               + bfc_ref[0])
        o_ref[...] = out[0]

    rep = lambda shape: pl.BlockSpec(shape, lambda i: tuple(0 for _ in shape))
    return pl.pallas_call(
        body,
        grid=grid,
        in_specs=[
            pl.BlockSpec((embT.shape[0], blk), lambda i: (0, i)),
            rep(w1t.shape),
            rep(b1c.shape),
            rep(w2t.shape),
            rep(b2c.shape),
            rep(w3t.shape),
            rep(b3c.shape),
            rep(wg.shape),
            rep(wh.shape),
            rep((1,)),
        ],
        out_specs=pl.BlockSpec((blk,), lambda i: (i,)),
        out_shape=jax.ShapeDtypeStruct((B,), jnp.float32),
    )(embT, w1t, b1c, w2t, b2c, w3t, b3c, wg, wh, bfc)


def kernel(x, table, W1, b1, W2, b2, W3, b3, Wfc, bfc):
    B = x.shape[0]
    x_flat = x.reshape(-1).astype(jnp.int32)
    out2d = _sc_gather_t(x_flat, _table_phys_flat(table))
    embT = (out2d.reshape(4, B // 128, 8, 128)
            .transpose(0, 2, 1, 3)
            .reshape(2 * EMBED_DIM, B))
    return _tc_mlp_t(
        embT,
        W1.T, b1.reshape(-1, 1),
        W2.T, b2.reshape(-1, 1),
        W3.T, b3.reshape(-1, 1),
        Wfc[:EMBED_DIM, :].T, Wfc[EMBED_DIM:, :].T, bfc,
    )


# speedup vs baseline: 19.5122x; 1.2039x over previous
import functools

import jax
import jax.numpy as jnp
from jax import lax
from jax.experimental import pallas as pl
from jax.experimental.pallas import tpu as pltpu
from jax.experimental.pallas import tpu_sc as plsc

EMBED_DIM = 16
NROWS = 2000000
FIELD_OFFSET = 1000000
NUM_WORKERS = 32
LANE_TILES = NROWS // 128
CHUNK = 128


def _table_phys_flat(table):
    t = table.T.reshape(2, 8, LANE_TILES, 128)
    return t.transpose(0, 2, 1, 3).reshape(-1)


def _sc_gather_t(u_ids, it_ids, tflat):
    B = u_ids.shape[0]
    per_w = B // NUM_WORKERS
    n_planes = 2 * EMBED_DIM
    n_words = per_w * n_planes
    n_chunks = n_words // CHUNK
    chunks_per_plane = per_w // CHUNK
    col_tiles = B // 128
    mesh = plsc.VectorSubcoreMesh(core_axis_name="c", subcore_axis_name="s")

    @functools.partial(
        pl.kernel,
        mesh=mesh,
        out_type=jax.ShapeDtypeStruct((B * n_planes // 128, 128), jnp.float32),
        compiler_params=pltpu.CompilerParams(
            use_tc_tiling_on_sc=False, needs_layout_passes=False),
        scratch_types=[
            pltpu.VMEM((2, per_w), jnp.int32),
            pltpu.VMEM((2, per_w), jnp.int32),
            pltpu.VMEM((n_words,), jnp.int32),
            pltpu.VMEM((n_planes, per_w), jnp.float32),
            pltpu.SemaphoreType.DMA,
        ],
    )
    def k(u_hbm, it_hbm, table_hbm, out_hbm, x_v, rp_v, w_idx, rows_v, sem):
        wid = lax.axis_index("s") * 2 + lax.axis_index("c")
        row0 = wid * per_w
        pltpu.sync_copy(u_hbm.at[pl.ds(row0, per_w)], x_v.at[0, :])
        pltpu.sync_copy(it_hbm.at[pl.ds(row0, per_w)], x_v.at[1, :])

        def build_rp(g, _):
            sl = pl.ds(g * 16, 16)
            for f in range(2):
                r16 = x_v[f, sl] + f * FIELD_OFFSET
                rp_v[f, sl] = ((r16 >> 7) << 10) + (r16 & 127)
            return 0

        lax.fori_loop(0, per_w // 16, build_rp, 0)

        def fire(c, _):
            p = c // chunks_per_plane
            b0 = (c % chunks_per_plane) * CHUNK
            f = p // EMBED_DIM
            j = p % EMBED_DIM
            jconst = (j // 8) * (LANE_TILES * 1024) + (j % 8) * 128
            for g in range(CHUNK // 16):
                sl = pl.ds(b0 + g * 16, 16)
                w_idx[pl.ds(c * CHUNK + g * 16, 16)] = rp_v[f, sl] + jconst
            pltpu.async_copy(
                table_hbm.at[w_idx.at[pl.ds(c * CHUNK, CHUNK)]],
                rows_v.at[p, pl.ds(b0, CHUNK)],
                sem,
            )
            return 0

        lax.fori_loop(0, n_chunks, fire, 0)
        for p in range(n_planes):
            pltpu.make_async_copy(
                table_hbm.at[pl.ds(0, per_w)], rows_v.at[p, :], sem
            ).wait()

        for R in range(n_planes // 8):
            for c in range(per_w // 128):
                t = R * col_tiles + wid * (per_w // 128) + c
                pltpu.sync_copy(
                    rows_v.at[pl.ds(8 * R, 8), pl.ds(128 * c, 128)],
                    out_hbm.at[pl.ds(8 * t, 8), :],
                )

    return k(u_ids, it_ids, tflat)


def _tc_mlp_t(embT, w1t, b1c, w2t, b2c, w3t, b3c, wg, wh, bfc):
    B = embT.shape[1]
    blk = 4096
    grid = (B // blk,)

    def body(e_ref, w1_ref, b1_ref, w2_ref, b2_ref, w3_ref, b3_ref,
             wg_ref, wh_ref, bfc_ref, o_ref):
        e = e_ref[...]
        h = jnp.maximum(
            jnp.dot(w1_ref[...], e, preferred_element_type=jnp.float32)
            + b1_ref[...], 0.0)
        h = jnp.maximum(
            jnp.dot(w2_ref[...], h, preferred_element_type=jnp.float32)
            + b2_ref[...], 0.0)
        h = jnp.maximum(
            jnp.dot(w3_ref[...], h, preferred_element_type=jnp.float32)
            + b3_ref[...], 0.0)
        gmf = e[:EMBED_DIM, :] * e[EMBED_DIM:, :]
        out = (jnp.dot(wg_ref[...], gmf, preferred_element_type=jnp.float32)
               + jnp.dot(wh_ref[...], h, preferred_element_type=jnp.float32)
               + bfc_ref[0])
        o_ref[...] = out[0]

    rep = lambda shape: pl.BlockSpec(shape, lambda i: tuple(0 for _ in shape))
    return pl.pallas_call(
        body,
        grid=grid,
        in_specs=[
            pl.BlockSpec((embT.shape[0], blk), lambda i: (0, i)),
            rep(w1t.shape),
            rep(b1c.shape),
            rep(w2t.shape),
            rep(b2c.shape),
            rep(w3t.shape),
            rep(b3c.shape),
            rep(wg.shape),
            rep(wh.shape),
            rep((1,)),
        ],
        out_specs=pl.BlockSpec((blk,), lambda i: (i,)),
        out_shape=jax.ShapeDtypeStruct((B,), jnp.float32),
    )(embT, w1t, b1c, w2t, b2c, w3t, b3c, wg, wh, bfc)


def kernel(x, table, W1, b1, W2, b2, W3, b3, Wfc, bfc):
    B = x.shape[0]
    x32 = x.astype(jnp.int32)
    out2d = _sc_gather_t(x32[:, 0], x32[:, 1], _table_phys_flat(table))
    embT = (out2d.reshape(4, B // 128, 8, 128)
            .transpose(0, 2, 1, 3)
            .reshape(2 * EMBED_DIM, B))
    return _tc_mlp_t(
        embT,
        W1.T, b1.reshape(-1, 1),
        W2.T, b2.reshape(-1, 1),
        W3.T, b3.reshape(-1, 1),
        Wfc[:EMBED_DIM, :].T, Wfc[EMBED_DIM:, :].T, bfc,
    )
